# baseline (speedup 1.0000x reference)
import functools

import jax
import jax.numpy as jnp
from jax import lax
from jax.experimental import pallas as pl
from jax.experimental.pallas import tpu as pltpu
from jax.experimental.pallas import tpu_sc as plsc

B = 16384
N_ROWS = 100000
Z_DIM = 64
ZP = 128
NC = 2
NS = 16
NW = NC * NS
BPW = B // NW
CHUNK = 128
NCHUNK = BPW // CHUNK

TBLK = 512
NBLK = (N_ROWS + TBLK - 1) // TBLK


def _tc_transpose_body(emb_t_ref, out_ref):
    out_ref[:, 0:Z_DIM] = emb_t_ref[...].T


_tc_relayout = pl.pallas_call(
    _tc_transpose_body,
    grid=(NBLK,),
    in_specs=[pl.BlockSpec((Z_DIM, TBLK), lambda i: (0, i))],
    out_specs=pl.BlockSpec((TBLK, ZP), lambda i: (i, 0)),
    out_shape=jax.ShapeDtypeStruct((N_ROWS, ZP), jnp.float32),
)


@functools.partial(
    pl.kernel,
    mesh=plsc.VectorSubcoreMesh(core_axis_name="c", subcore_axis_name="s"),
    out_type=(
        jax.ShapeDtypeStruct((B, ZP), jnp.float32),
        jax.ShapeDtypeStruct((B,), jnp.float32),
        jax.ShapeDtypeStruct((B,), jnp.float32),
    ),
    scratch_types=[
        pltpu.VMEM((BPW,), jnp.int32),
        pltpu.VMEM((BPW, ZP), jnp.float32),
        pltpu.VMEM((BPW,), jnp.float32),
        pltpu.VMEM((BPW,), jnp.float32),
        pltpu.SemaphoreType.DMA,
    ],
    compiler_params=pltpu.CompilerParams(use_tc_tiling_on_sc=True),
)
def _sc_gather(idx_hbm, emb_hbm, inh_hbm, det_hbm,
               z_out, inh_out, det_out,
               idx_v, z_v, inh_v, det_v, sem):
    wid = lax.axis_index("s") * NC + lax.axis_index("c")
    base = wid * BPW
    pltpu.sync_copy(idx_hbm.at[pl.ds(base, BPW)], idx_v)
    copies = []
    for c in range(NCHUNK):
        sl = pl.ds(c * CHUNK, CHUNK)
        copies.append(pltpu.async_copy(emb_hbm.at[idx_v.at[sl]], z_v.at[sl], sem))
        copies.append(pltpu.async_copy(inh_hbm.at[idx_v.at[sl]], inh_v.at[sl], sem))
        copies.append(pltpu.async_copy(det_hbm.at[idx_v.at[sl]], det_v.at[sl], sem))
    for cp in copies:
        cp.wait()
    pltpu.sync_copy(z_v, z_out.at[pl.ds(base, BPW)])
    pltpu.sync_copy(inh_v, inh_out.at[pl.ds(base, BPW)])
    pltpu.sync_copy(det_v, det_out.at[pl.ds(base, BPW)])


def kernel(piano_model, emb_w, inharm_w, detune_w):
    idx = piano_model.astype(jnp.int32)
    emb128 = _tc_relayout(emb_w.T)
    z128, inh, det = _sc_gather(idx, emb128,
                                inharm_w.reshape(-1), detune_w.reshape(-1))
    return (z128[:, None, :Z_DIM],
            inh.reshape(B, 1, 1),
            det.reshape(B, 1, 1))


# speedup vs baseline: 1.5481x; 1.5481x over previous
import functools

import jax
import jax.numpy as jnp
from jax import lax
from jax.experimental import pallas as pl
from jax.experimental.pallas import tpu as pltpu
from jax.experimental.pallas import tpu_sc as plsc

B = 16384
N_ROWS = 100000
Z_DIM = 64
ZP = 128
NC = 2
NS = 16
NW = NC * NS
BPW = B // NW
CHUNK = 128
NCHUNK = BPW // CHUNK

TBLK = 2048
NBLK = (N_ROWS + TBLK - 1) // TBLK


def _tc_transpose_body(emb_t_ref, out_ref):
    x = emb_t_ref[...]
    row = jax.lax.broadcasted_iota(jnp.int32, (Z_DIM, Z_DIM), 0)
    col = jax.lax.broadcasted_iota(jnp.int32, (Z_DIM, Z_DIM), 1)
    eye = (row == col).astype(jnp.float32)
    out_ref[:, 0:Z_DIM] = jax.lax.dot_general(
        x, eye, (((0,), (0,)), ((), ())),
        preferred_element_type=jnp.float32,
        precision=jax.lax.Precision.HIGHEST)


_tc_relayout = pl.pallas_call(
    _tc_transpose_body,
    grid=(NBLK,),
    in_specs=[pl.BlockSpec((Z_DIM, TBLK), lambda i: (0, i))],
    out_specs=pl.BlockSpec((TBLK, ZP), lambda i: (i, 0)),
    out_shape=jax.ShapeDtypeStruct((N_ROWS, ZP), jnp.float32),
)


@functools.partial(
    pl.kernel,
    mesh=plsc.VectorSubcoreMesh(core_axis_name="c", subcore_axis_name="s"),
    out_type=(
        jax.ShapeDtypeStruct((B, ZP), jnp.float32),
        jax.ShapeDtypeStruct((B,), jnp.float32),
        jax.ShapeDtypeStruct((B,), jnp.float32),
    ),
    scratch_types=[
        pltpu.VMEM((BPW,), jnp.int32),
        pltpu.VMEM((BPW, ZP), jnp.float32),
        pltpu.VMEM((BPW,), jnp.float32),
        pltpu.VMEM((BPW,), jnp.float32),
        pltpu.SemaphoreType.DMA,
    ],
    compiler_params=pltpu.CompilerParams(use_tc_tiling_on_sc=True),
)
def _sc_gather(idx_hbm, emb_hbm, inh_hbm, det_hbm,
               z_out, inh_out, det_out,
               idx_v, z_v, inh_v, det_v, sem):
    wid = lax.axis_index("s") * NC + lax.axis_index("c")
    base = wid * BPW
    pltpu.sync_copy(idx_hbm.at[pl.ds(base, BPW)], idx_v)
    copies = []
    for c in range(NCHUNK):
        sl = pl.ds(c * CHUNK, CHUNK)
        copies.append(pltpu.async_copy(emb_hbm.at[idx_v.at[sl]], z_v.at[sl], sem))
        copies.append(pltpu.async_copy(inh_hbm.at[idx_v.at[sl]], inh_v.at[sl], sem))
        copies.append(pltpu.async_copy(det_hbm.at[idx_v.at[sl]], det_v.at[sl], sem))
    for cp in copies:
        cp.wait()
    pltpu.sync_copy(z_v, z_out.at[pl.ds(base, BPW)])
    pltpu.sync_copy(inh_v, inh_out.at[pl.ds(base, BPW)])
    pltpu.sync_copy(det_v, det_out.at[pl.ds(base, BPW)])


def kernel(piano_model, emb_w, inharm_w, detune_w):
    idx = piano_model.astype(jnp.int32)
    emb128 = _tc_relayout(emb_w.T)
    z128, inh, det = _sc_gather(idx, emb128,
                                inharm_w.reshape(-1), detune_w.reshape(-1))
    return (z128[:, None, :Z_DIM],
            inh.reshape(B, 1, 1),
            det.reshape(B, 1, 1))


# speedup vs baseline: 1.7902x; 1.1564x over previous
import functools

import jax
import jax.numpy as jnp
from jax import lax
from jax.experimental import pallas as pl
from jax.experimental.pallas import tpu as pltpu
from jax.experimental.pallas import tpu_sc as plsc

B = 16384
Z_DIM = 64
NC = 2
NS = 16
NW = NC * NS
BPW = B // NW
CHUNK = 128
NCHUNK = BPW // CHUNK
L = 16


@functools.partial(
    pl.kernel,
    mesh=plsc.VectorSubcoreMesh(core_axis_name="c", subcore_axis_name="s"),
    out_type=(
        jax.ShapeDtypeStruct((B, Z_DIM), jnp.float32),
        jax.ShapeDtypeStruct((B,), jnp.float32),
        jax.ShapeDtypeStruct((B,), jnp.float32),
    ),
    scratch_types=[
        pltpu.VMEM((BPW,), jnp.int32),
        pltpu.VMEM((BPW, Z_DIM), jnp.float32),
        pltpu.VMEM((BPW,), jnp.float32),
        pltpu.VMEM((BPW,), jnp.float32),
        pltpu.SemaphoreType.DMA,
        pltpu.SemaphoreType.DMA,
    ],
    compiler_params=pltpu.CompilerParams(use_tc_tiling_on_sc=True),
)
def _sc_gather(idx_hbm, emb_hbm, inh_hbm, det_hbm,
               z_out, inh_out, det_out,
               idx_v, z_v, inh_v, det_v, sem, row_sem):
    wid = lax.axis_index("s") * NC + lax.axis_index("c")
    base = wid * BPW
    pltpu.sync_copy(idx_hbm.at[pl.ds(base, BPW)], idx_v)
    copies = []
    for c in range(NCHUNK):
        sl = pl.ds(c * CHUNK, CHUNK)
        copies.append(pltpu.async_copy(inh_hbm.at[idx_v.at[sl]], inh_v.at[sl], sem))
        copies.append(pltpu.async_copy(det_hbm.at[idx_v.at[sl]], det_v.at[sl], sem))

    def body(g, carry):
        vec = idx_v[pl.ds(g * L, L)]
        cps = []
        for t in range(L):
            j = g * L + t
            cps.append(pltpu.async_copy(
                emb_hbm.at[pl.ds(vec[t], 1)], z_v.at[pl.ds(j, 1)], row_sem))
        for cp in cps:
            cp.wait()
        return carry

    lax.fori_loop(0, BPW // L, body, 0)
    for cp in copies:
        cp.wait()
    pltpu.sync_copy(z_v, z_out.at[pl.ds(base, BPW)])
    pltpu.sync_copy(inh_v, inh_out.at[pl.ds(base, BPW)])
    pltpu.sync_copy(det_v, det_out.at[pl.ds(base, BPW)])


def kernel(piano_model, emb_w, inharm_w, detune_w):
    idx = piano_model.astype(jnp.int32)
    z, inh, det = _sc_gather(idx, emb_w,
                             inharm_w.reshape(-1), detune_w.reshape(-1))
    return (z[:, None, :],
            inh.reshape(B, 1, 1),
            det.reshape(B, 1, 1))


# speedup vs baseline: 2.2171x; 1.2384x over previous
import functools

import jax
import jax.numpy as jnp
from jax import lax
from jax.experimental import pallas as pl
from jax.experimental.pallas import tpu as pltpu
from jax.experimental.pallas import tpu_sc as plsc

B = 16384
Z_DIM = 64
NC = 2
NS = 16
NW = NC * NS
BPW = B // NW
CHUNK = 128
NCHUNK = BPW // CHUNK
L = 16


@functools.partial(
    pl.kernel,
    mesh=plsc.VectorSubcoreMesh(core_axis_name="c", subcore_axis_name="s"),
    out_type=(
        jax.ShapeDtypeStruct((B, Z_DIM), jnp.float32),
        jax.ShapeDtypeStruct((B,), jnp.float32),
        jax.ShapeDtypeStruct((B,), jnp.float32),
    ),
    scratch_types=[
        pltpu.VMEM((BPW,), jnp.int32),
        pltpu.VMEM((BPW, Z_DIM), jnp.float32),
        pltpu.VMEM((BPW,), jnp.float32),
        pltpu.VMEM((BPW,), jnp.float32),
        pltpu.SemaphoreType.DMA,
        pltpu.SemaphoreType.DMA,
    ],
    compiler_params=pltpu.CompilerParams(use_tc_tiling_on_sc=True),
)
def _sc_gather(idx_hbm, emb_hbm, inh_hbm, det_hbm,
               z_out, inh_out, det_out,
               idx_v, z_v, inh_v, det_v, sem, row_sem):
    wid = lax.axis_index("s") * NC + lax.axis_index("c")
    base = wid * BPW
    pltpu.sync_copy(idx_hbm.at[pl.ds(base, BPW)], idx_v)
    copies = []
    for c in range(NCHUNK):
        sl = pl.ds(c * CHUNK, CHUNK)
        copies.append(pltpu.async_copy(inh_hbm.at[idx_v.at[sl]], inh_v.at[sl], sem))
        copies.append(pltpu.async_copy(det_hbm.at[idx_v.at[sl]], det_v.at[sl], sem))

    def body(g, carry):
        vec = idx_v[pl.ds(g * L, L)]
        for t in range(L):
            j = g * L + t
            pltpu.async_copy(
                emb_hbm.at[pl.ds(vec[t], 1)], z_v.at[pl.ds(j, 1)], row_sem)
        return carry

    lax.fori_loop(0, BPW // L, body, 0)
    pltpu.make_async_copy(emb_hbm.at[pl.ds(0, BPW)], z_v, row_sem).wait()
    for cp in copies:
        cp.wait()
    pltpu.sync_copy(z_v, z_out.at[pl.ds(base, BPW)])
    pltpu.sync_copy(inh_v, inh_out.at[pl.ds(base, BPW)])
    pltpu.sync_copy(det_v, det_out.at[pl.ds(base, BPW)])


def kernel(piano_model, emb_w, inharm_w, detune_w):
    idx = piano_model.astype(jnp.int32)
    z, inh, det = _sc_gather(idx, emb_w,
                             inharm_w.reshape(-1), detune_w.reshape(-1))
    return (z[:, None, :],
            inh.reshape(B, 1, 1),
            det.reshape(B, 1, 1))
